# baseline (speedup 1.0000x reference)
import functools

import jax
import jax.numpy as jnp
from jax import lax
from jax.experimental import pallas as pl
from jax.experimental.pallas import tpu as pltpu
from jax.experimental.pallas import tpu_sc as plsc

_N_CLS = 1000000
_P = 0.1
_CHUNK = 128


@functools.cache
def _make_gather(V, D, B):
    info = plsc.get_sparse_core_info()
    nc, ns = info.num_cores, info.num_subcores
    nw = nc * ns
    b_per_w = B // nw
    n_chunks = b_per_w // _CHUNK
    assert b_per_w % _CHUNK == 0 and B % nw == 0

    mesh = plsc.VectorSubcoreMesh(core_axis_name="c", subcore_axis_name="s")

    @functools.partial(
        pl.kernel,
        mesh=mesh,
        compiler_params=pltpu.CompilerParams(use_tc_tiling_on_sc=False),
        out_type=jax.ShapeDtypeStruct((B, D), jnp.float32),
        scratch_types=[
            pltpu.VMEM((n_chunks, _CHUNK), jnp.int32),
            pltpu.VMEM((n_chunks, _CHUNK, D), jnp.float32),
            pltpu.SemaphoreType.DMA,
        ],
    )
    def gather_kernel(idx_hbm, table_hbm, out_hbm, idx_v, rows_v, sem):
        wid = lax.axis_index("s") * nc + lax.axis_index("c")
        base = wid * b_per_w
        pltpu.sync_copy(idx_hbm.at[wid], idx_v)
        copies = [
            pltpu.async_copy(table_hbm.at[idx_v.at[j]], rows_v.at[j], sem)
            for j in range(n_chunks)
        ]
        for j, c in enumerate(copies):
            c.wait()
            pltpu.sync_copy(
                rows_v.at[j], out_hbm.at[pl.ds(base + j * _CHUNK, _CHUNK)]
            )

    return gather_kernel


def kernel(y, train, table):
    mask = jax.random.uniform(jax.random.key(1), y.shape, dtype=jnp.float32) < _P
    yy = jnp.where(
        jnp.logical_and(train, mask), jnp.full_like(y, _N_CLS), y
    ).astype(jnp.int32)
    V, D = table.shape
    B = y.shape[0]
    info = plsc.get_sparse_core_info()
    nw = info.num_cores * info.num_subcores
    idx3 = yy.reshape(nw, (B // nw) // _CHUNK, _CHUNK)
    g = _make_gather(V, D, B)
    return g(idx3, table)


# speedup vs baseline: 1.7086x; 1.7086x over previous
import functools

import jax
import jax.numpy as jnp
from jax import lax
from jax.experimental import pallas as pl
from jax.experimental.pallas import tpu as pltpu
from jax.experimental.pallas import tpu_sc as plsc

_N_CLS = 1000000
_P = 0.1


@functools.cache
def _make_gather(V, D, B):
    info = plsc.get_sparse_core_info()
    nc, ns = info.num_cores, info.num_subcores
    nw = nc * ns
    b_per_w = B // nw
    assert B % nw == 0

    mesh = plsc.VectorSubcoreMesh(core_axis_name="c", subcore_axis_name="s")

    @functools.partial(
        pl.kernel,
        mesh=mesh,
        out_type=jax.ShapeDtypeStruct((B, D), jnp.float32),
        scratch_types=[
            pltpu.VMEM((b_per_w,), jnp.int32),
            pltpu.VMEM((b_per_w, D), jnp.float32),
            pltpu.SemaphoreType.DMA,
        ],
    )
    def gather_kernel(idx_hbm, table_hbm, out_hbm, idx_vm, rows_v, sem):
        wid = lax.axis_index("s") * nc + lax.axis_index("c")
        base = wid * b_per_w
        pltpu.sync_copy(idx_hbm.at[wid], idx_vm)

        def body(i, carry):
            v = idx_vm[pl.ds(i * 16, 16)]
            for lane in range(16):
                r = v[lane]
                pltpu.async_copy(
                    table_hbm.at[pl.ds(r, 1)],
                    rows_v.at[pl.ds(i * 16 + lane, 1)],
                    sem,
                )
            return carry

        lax.fori_loop(0, b_per_w // 16, body, 0)
        pltpu.make_async_copy(table_hbm.at[pl.ds(0, b_per_w)], rows_v, sem).wait()
        pltpu.sync_copy(rows_v, out_hbm.at[pl.ds(base, b_per_w)])

    return gather_kernel


def kernel(y, train, table):
    mask = jax.random.uniform(jax.random.key(1), y.shape, dtype=jnp.float32) < _P
    yy = jnp.where(
        jnp.logical_and(train, mask), jnp.full_like(y, _N_CLS), y
    ).astype(jnp.int32)
    V, D = table.shape
    B = y.shape[0]
    info = plsc.get_sparse_core_info()
    nw = info.num_cores * info.num_subcores
    idx2 = yy.reshape(nw, B // nw)
    g = _make_gather(V, D, B)
    return g(idx2, table)


# speedup vs baseline: 1.7191x; 1.0062x over previous
import functools

import jax
import jax.numpy as jnp
from jax import lax
from jax.experimental import pallas as pl
from jax.experimental.pallas import tpu as pltpu
from jax.experimental.pallas import tpu_sc as plsc

_N_CLS = 1000000
_P = 0.1


@functools.cache
def _make_gather(V, D, B):
    info = plsc.get_sparse_core_info()
    nc, ns, L = info.num_cores, info.num_subcores, info.num_lanes
    nw = nc * ns
    b_per_w = B // nw
    assert B % nw == 0 and b_per_w % L == 0

    mesh = plsc.VectorSubcoreMesh(core_axis_name="c", subcore_axis_name="s")

    @functools.partial(
        pl.kernel,
        mesh=mesh,
        out_type=jax.ShapeDtypeStruct((B, D), jnp.float32),
        scratch_types=[
            pltpu.VMEM((b_per_w,), jnp.int32),
            pltpu.VMEM((b_per_w, D), jnp.float32),
            pltpu.SemaphoreType.DMA,
        ],
    )
    def gather_kernel(idx_hbm, table_hbm, out_hbm, idx_vm, rows_v, sem):
        wid = lax.axis_index("s") * nc + lax.axis_index("c")
        base = wid * b_per_w
        pltpu.sync_copy(idx_hbm.at[wid], idx_vm)

        def body(i, carry):
            v = idx_vm[pl.ds(i * L, L)]
            for lane in range(L):
                r = v[lane]
                pltpu.async_copy(
                    table_hbm.at[pl.ds(r, 1)],
                    rows_v.at[pl.ds(i * L + lane, 1)],
                    sem,
                )
            return carry

        lax.fori_loop(0, b_per_w // L, body, 0)
        pltpu.make_async_copy(table_hbm.at[pl.ds(0, b_per_w)], rows_v, sem).wait()
        pltpu.sync_copy(rows_v, out_hbm.at[pl.ds(base, b_per_w)])

    return gather_kernel


def kernel(y, train, table):
    mask = jax.random.uniform(jax.random.key(1), y.shape, dtype=jnp.float32) < _P
    yy = jnp.where(
        jnp.logical_and(train, mask), jnp.full_like(y, _N_CLS), y
    ).astype(jnp.int32)
    V, D = table.shape
    B = y.shape[0]
    info = plsc.get_sparse_core_info()
    nw = info.num_cores * info.num_subcores
    idx2 = yy.reshape(nw, B // nw)
    g = _make_gather(V, D, B)
    return g(idx2, table)
